# Initial kernel scaffold; baseline (speedup 1.0000x reference)
import jax
import jax.numpy as jnp
from jax.experimental import pallas as pl


def kernel(x, memory):
    raise NotImplementedError("write your pallas kernel here")


# speedup vs baseline: 13.3325x; 13.3325x over previous
import functools

import jax
import jax.numpy as jnp
from jax import lax
from jax.experimental import pallas as pl
from jax.experimental.pallas import tpu as pltpu
from jax.experimental.pallas import tpu_sc as plsc

B, S, D = 8, 1024, 64
K = 8192
TOPK = 5
KPAD = 8
KC = 10
KCPAD = 16
MAX_ITERS = 10
NEG = float("-inf")


BK = 256
NB = K // BK


def _stage1_body(x_ref, mem_ref, idx_ref, sim_ref):
    x = x_ref[0]

    def mm_body(j, carry):
        m = mem_ref[pl.ds(j * BK, BK), :]
        sim_ref[:, pl.ds(j * BK, BK)] = lax.dot_general(
            x, m, (((1,), (1,)), ((), ())),
            preferred_element_type=jnp.float32)
        return carry
    lax.fori_loop(0, NB, mm_body, 0)

    biota = lax.broadcasted_iota(jnp.int32, (S, BK), 1)
    vals = []
    idxs = []
    vk = None
    ik = None
    for k in range(TOPK):
        def pass_body(j, carry, vk=vk, ik=ik):
            v, i = carry
            blk = sim_ref[:, pl.ds(j * BK, BK)]
            gidx = biota + j * BK
            if k == 0:
                cand = blk
            else:
                keep = (blk < vk) | ((blk == vk) & (gidx > ik))
                cand = jnp.where(keep, blk, NEG)
            bv = jnp.max(cand, axis=1, keepdims=True)
            bi = jnp.min(jnp.where(cand == bv, gidx, K), axis=1,
                         keepdims=True)
            better = bv > v
            return jnp.where(better, bv, v), jnp.where(better, bi, i)
        v0 = jnp.full((S, 1), NEG, jnp.float32)
        i0 = jnp.full((S, 1), K, jnp.int32)
        v, i = lax.fori_loop(0, NB, pass_body, (v0, i0))
        vals.append(v)
        idxs.append(i)
        vk, ik = v, i

    del vals
    for k in range(TOPK):
        idx_ref[0, :, k:k + 1] = idxs[k]
    for k in range(TOPK, KPAD):
        idx_ref[0, :, k:k + 1] = jnp.zeros((S, 1), jnp.int32)


def _stage1(x, memory):
    return pl.pallas_call(
        _stage1_body,
        grid=(B,),
        in_specs=[
            pl.BlockSpec((1, S, D), lambda b: (b, 0, 0)),
            pl.BlockSpec((K, D), lambda b: (0, 0)),
        ],
        out_specs=pl.BlockSpec((1, S, KPAD), lambda b: (b, 0, 0)),
        out_shape=jax.ShapeDtypeStruct((B, S, KPAD), jnp.int32),
        scratch_shapes=[pltpu.VMEM((S, K), jnp.float32)],
        compiler_params=pltpu.CompilerParams(
            vmem_limit_bytes=100 * 1024 * 1024),
    )(x, memory)


_NW = 32
_TOTAL = B * S * KPAD
_PER_W = _TOTAL // _NW
_CHUNK = 1024


def _sc_gather_body(table_hbm, idx_hbm, out_hbm, idx_v, rows_v, sem):
    wid = lax.axis_index("s") * 2 + lax.axis_index("c")
    for c in range(_PER_W // _CHUNK):
        base = wid * _PER_W + c * _CHUNK
        pltpu.sync_copy(idx_hbm.at[pl.ds(base, _CHUNK)], idx_v)
        pltpu.async_copy(table_hbm.at[idx_v], rows_v, sem).wait()
        pltpu.sync_copy(rows_v, out_hbm.at[pl.ds(base, _CHUNK)])


def _stage2(memory, idx_flat):
    mesh = plsc.VectorSubcoreMesh(core_axis_name="c", subcore_axis_name="s")
    k = functools.partial(
        pl.kernel,
        mesh=mesh,
        out_type=jax.ShapeDtypeStruct((_TOTAL, D), jnp.float32),
        scratch_types=[
            pltpu.VMEM((_CHUNK,), jnp.int32),
            pltpu.VMEM((_CHUNK, D), jnp.float32),
            pltpu.SemaphoreType.DMA,
        ],
        compiler_params=pltpu.CompilerParams(use_tc_tiling_on_sc=False),
    )(_sc_gather_body)
    return k(memory, idx_flat)


def _stage3_body(x_ref, ctx_ref, init_ref, out_ref, xe_ref):
    x = x_ref[0]
    ctxs = [ctx_ref[0][:, k, :] for k in range(TOPK)]
    logits = [jnp.sum(x * c, axis=1, keepdims=True) for c in ctxs]
    mx = logits[0]
    for t in logits[1:]:
        mx = jnp.maximum(mx, t)
    es = [jnp.exp(t - mx) for t in logits]
    denom = es[0]
    for e in es[1:]:
        denom = denom + e
    acc = (es[0] / denom) * ctxs[0]
    for k in range(1, TOPK):
        acc = acc + (es[k] / denom) * ctxs[k]
    xe = x + acc
    xe_ref[...] = xe

    cent_rows = []
    for j in range(KC):
        cent_rows.append(xe_ref[pl.ds(init_ref[0, 0, j], 1), :])
    for j in range(KC, KCPAD):
        cent_rows.append(jnp.zeros((1, D), jnp.float32))
    cent = jnp.concatenate(cent_rows, axis=0)

    xn = jnp.sum(xe * xe, axis=1, keepdims=True)
    lane16 = lax.broadcasted_iota(jnp.int32, (S, KCPAD), 1)
    valid = lane16 < KC
    assign_oh = None
    xe_b = xe.astype(jnp.bfloat16)
    for _ in range(MAX_ITERS):
        dots = lax.dot_general(xe_b, cent.astype(jnp.bfloat16),
                               (((1,), (1,)), ((), ())),
                               preferred_element_type=jnp.float32)
        cn = jnp.sum(cent * cent, axis=1)
        dist = (xn - 2.0 * dots) + cn[None, :]
        dist = jnp.where(valid, dist, float("inf"))
        m = jnp.min(dist, axis=1, keepdims=True)
        a = jnp.min(jnp.where(dist == m, lane16, KCPAD), axis=1, keepdims=True)
        assign_oh = (lane16 == a).astype(jnp.float32)
        sums = lax.dot_general(assign_oh, xe, (((0,), (0,)), ((), ())),
                               preferred_element_type=jnp.float32,
                               precision=lax.Precision.HIGHEST)
        counts = jnp.sum(assign_oh, axis=0)
        cent = sums / (counts[:, None] + 1e-08)

    out_ref[0] = lax.dot_general(assign_oh, cent, (((1,), (0,)), ((), ())),
                                 preferred_element_type=jnp.float32,
                                 precision=lax.Precision.HIGHEST)


def _stage3(x, ctx, init_idx):
    return pl.pallas_call(
        _stage3_body,
        grid=(B,),
        in_specs=[
            pl.BlockSpec((1, S, D), lambda b: (b, 0, 0)),
            pl.BlockSpec((1, S, KPAD, D), lambda b: (b, 0, 0, 0)),
            pl.BlockSpec((1, 1, KC), lambda b: (b, 0, 0), memory_space=pltpu.SMEM),
        ],
        out_specs=pl.BlockSpec((1, S, D), lambda b: (b, 0, 0)),
        out_shape=jax.ShapeDtypeStruct((B, S, D), jnp.float32),
        scratch_shapes=[pltpu.VMEM((S, D), jnp.float32)],
    )(x, ctx, init_idx)


def kernel(x, memory):
    idx = _stage1(x, memory)
    ctx = _stage2(memory, idx.reshape(_TOTAL))
    ctx = ctx.reshape(B, S, KPAD, D)
    init_idx = jax.random.randint(jax.random.key(42), (B, KC), 0, S).astype(jnp.int32)
    init_idx = init_idx.reshape(B, 1, KC)
    return _stage3(x, ctx, init_idx)
